# baseline (device time: 1446286 ns/iter reference)
import jax
import jax.numpy as jnp
from jax import lax
from jax.experimental import pallas as pl
from jax.experimental.pallas import tpu as pltpu

jax.config.update("jax_compilation_cache_dir", "/tmp/jaxcache")
jax.config.update("jax_persistent_cache_min_compile_time_secs", 1.0)

N_DEV = 4
NB = 2


def kernel(x, w_mat):
    x = x.astype(jnp.bfloat16)
    w_mat = w_mat.astype(jnp.bfloat16)
    m, k_shard = x.shape
    _, n = w_mat.shape
    chunk = m // N_DEV
    band = chunk // NB
    h = n // 2

    def body(x_hbm, w_hbm, out_hbm, w_vmem, x_vmem,
             send_r, recv_r, send_l, recv_l, local_sem,
             rsr_send, rsr_recv, rsl_send, rsl_recv,
             agr_send, agr_recv, agl_send, agl_recv,
             credit_r, credit_l):
        j = pl.program_id(0)
        r = lax.axis_index("i")
        right = lax.rem(r + 1, N_DEV)
        left = lax.rem(r + N_DEV - 1, N_DEV)

        @pl.when(j == 0)
        def _():
            bsem = pltpu.get_barrier_semaphore()
            pl.semaphore_signal(bsem, inc=1, device_id=(left,),
                                device_id_type=pl.DeviceIdType.MESH)
            pl.semaphore_signal(bsem, inc=1, device_id=(right,),
                                device_id_type=pl.DeviceIdType.MESH)
            pl.semaphore_wait(bsem, 2)
            cw = pltpu.make_async_copy(w_hbm, w_vmem, local_sem)
            cw.start()
            cw.wait()

        def load_x(c):
            cp = pltpu.make_async_copy(
                x_hbm.at[pl.ds(c * chunk + j * band, band), :], x_vmem,
                local_sem)
            cp.start()
            cp.wait()

        def partial_r(c):
            load_x(c)
            return jnp.dot(x_vmem[...], w_vmem[:, 0:h],
                           preferred_element_type=jnp.float32)

        def partial_l(c):
            load_x(c)
            return jnp.dot(x_vmem[...], w_vmem[:, h:n],
                           preferred_element_type=jnp.float32)

        send_r[...] = partial_r(lax.rem(r + N_DEV - 1, N_DEV))
        send_l[...] = partial_l(lax.rem(r + 1, N_DEV))

        def rs_step(s, carry):
            @pl.when(jnp.logical_or(j > 0, s > 0))
            def _():
                pl.semaphore_wait(credit_r, 1)
                pl.semaphore_wait(credit_l, 1)

            q = h // 2
            rdmas = []
            for qi in range(2):
                cs = slice(qi * q, (qi + 1) * q)
                rdmas.append(pltpu.make_async_remote_copy(
                    src_ref=send_r.at[:, cs], dst_ref=recv_r.at[:, cs],
                    send_sem=rsr_send.at[j, s, qi],
                    recv_sem=rsr_recv.at[j, s, qi],
                    device_id=(right,),
                    device_id_type=pl.DeviceIdType.MESH))
                rdmas.append(pltpu.make_async_remote_copy(
                    src_ref=send_l.at[:, cs], dst_ref=recv_l.at[:, cs],
                    send_sem=rsl_send.at[j, s, qi],
                    recv_sem=rsl_recv.at[j, s, qi],
                    device_id=(left,),
                    device_id_type=pl.DeviceIdType.MESH))
            for rd in rdmas:
                rd.start()
            for rd in rdmas:
                rd.wait()

            cr = lax.rem(r - 2 - s + 2 * N_DEV, N_DEV)
            send_r[...] = partial_r(cr) + recv_r[...]
            cl = lax.rem(r + 2 + s, N_DEV)
            send_l[...] = partial_l(cl) + recv_l[...]

            @pl.when(jnp.logical_not(
                jnp.logical_and(j == NB - 1, s == N_DEV - 2)))
            def _():
                pl.semaphore_signal(credit_r, inc=1, device_id=(left,),
                                    device_id_type=pl.DeviceIdType.MESH)
                pl.semaphore_signal(credit_l, inc=1, device_id=(right,),
                                    device_id_type=pl.DeviceIdType.MESH)
            return carry

        lax.fori_loop(0, N_DEV - 1, rs_step, 0)

        c1 = pltpu.make_async_copy(
            send_r, out_hbm.at[pl.ds(r * chunk + j * band, band), 0:h],
            local_sem)
        c1.start()
        c2 = pltpu.make_async_copy(
            send_l, out_hbm.at[pl.ds(r * chunk + j * band, band), h:n],
            local_sem)
        c2.start()
        c1.wait()
        c2.wait()

        def ag_step(t, carry):
            ar = lax.rem(r - t + N_DEV, N_DEV)
            al = lax.rem(r + t, N_DEV)
            q = h // 2
            rdmas = []
            for qi in range(2):
                srcr = out_hbm.at[pl.ds(ar * chunk + j * band, band),
                                  pl.ds(qi * q, q)]
                rdmas.append(pltpu.make_async_remote_copy(
                    src_ref=srcr, dst_ref=srcr,
                    send_sem=agr_send.at[j, t, qi],
                    recv_sem=agr_recv.at[j, t, qi],
                    device_id=(right,),
                    device_id_type=pl.DeviceIdType.MESH))
                srcl = out_hbm.at[pl.ds(al * chunk + j * band, band),
                                  pl.ds(h + qi * q, q)]
                rdmas.append(pltpu.make_async_remote_copy(
                    src_ref=srcl, dst_ref=srcl,
                    send_sem=agl_send.at[j, t, qi],
                    recv_sem=agl_recv.at[j, t, qi],
                    device_id=(left,),
                    device_id_type=pl.DeviceIdType.MESH))
            for rd in rdmas:
                rd.start()
            for rd in rdmas:
                rd.wait()
            return carry

        lax.fori_loop(0, N_DEV - 1, ag_step, 0)

    nsteps = (NB, N_DEV - 1, 2)
    return pl.pallas_call(
        body,
        grid=(NB,),
        out_shape=jax.ShapeDtypeStruct((m, n), jnp.float32),
        in_specs=[pl.BlockSpec(memory_space=pl.ANY),
                  pl.BlockSpec(memory_space=pl.ANY)],
        out_specs=pl.BlockSpec(memory_space=pl.ANY),
        scratch_shapes=[
            pltpu.VMEM((k_shard, n), jnp.bfloat16),
            pltpu.VMEM((band, k_shard), jnp.bfloat16),
            pltpu.VMEM((band, h), jnp.float32),
            pltpu.VMEM((band, h), jnp.float32),
            pltpu.VMEM((band, h), jnp.float32),
            pltpu.VMEM((band, h), jnp.float32),
            pltpu.SemaphoreType.DMA,
            pltpu.SemaphoreType.DMA(nsteps),
            pltpu.SemaphoreType.DMA(nsteps),
            pltpu.SemaphoreType.DMA(nsteps),
            pltpu.SemaphoreType.DMA(nsteps),
            pltpu.SemaphoreType.DMA(nsteps),
            pltpu.SemaphoreType.DMA(nsteps),
            pltpu.SemaphoreType.DMA(nsteps),
            pltpu.SemaphoreType.DMA(nsteps),
            pltpu.SemaphoreType.REGULAR,
            pltpu.SemaphoreType.REGULAR,
        ],
        compiler_params=pltpu.CompilerParams(
            collective_id=0, vmem_limit_bytes=63 * 1024 * 1024,
            dimension_semantics=("arbitrary",)),
    )(x, w_mat)


# device time: 846110 ns/iter; 1.7093x vs baseline; 1.7093x over previous
import jax
import jax.numpy as jnp
from jax import lax
from jax.experimental import pallas as pl
from jax.experimental.pallas import tpu as pltpu

jax.config.update("jax_compilation_cache_dir", "/tmp/jaxcache")
jax.config.update("jax_persistent_cache_min_compile_time_secs", 1.0)

N_DEV = 4
NB = 4


def kernel(x, w_mat):
    x = x.astype(jnp.bfloat16)
    w_mat = w_mat.astype(jnp.bfloat16)
    m, k_shard = x.shape
    _, n = w_mat.shape
    chunk = m // N_DEV
    band = chunk // NB
    h = n // 2

    def body(x_hbm, w_hbm, out_hbm, w_vmem, x_vmem,
             send_r, recv_r, send_l, recv_l, tmp_r, tmp_l, ag_r, ag_l,
             sem_a, sem_b,
             rsr_send, rsr_recv, rsl_send, rsl_recv,
             agr_send, agr_recv, agl_send, agl_recv,
             credit_r, credit_l):
        j = pl.program_id(0)
        r = lax.axis_index("i")
        right = lax.rem(r + 1, N_DEV)
        left = lax.rem(r + N_DEV - 1, N_DEV)

        @pl.when(j == 0)
        def _():
            bsem = pltpu.get_barrier_semaphore()
            pl.semaphore_signal(bsem, inc=1, device_id=(left,),
                                device_id_type=pl.DeviceIdType.MESH)
            pl.semaphore_signal(bsem, inc=1, device_id=(right,),
                                device_id_type=pl.DeviceIdType.MESH)
            pl.semaphore_wait(bsem, 2)
            cw = pltpu.make_async_copy(w_hbm, w_vmem, sem_a)
            cw.start()
            cw.wait()

        def load_x(c):
            cp = pltpu.make_async_copy(
                x_hbm.at[pl.ds(c * chunk + j * band, band), :], x_vmem,
                sem_a)
            cp.start()
            cp.wait()

        def partial_r(c):
            load_x(c)
            return jnp.dot(x_vmem[...], w_vmem[:, 0:h],
                           preferred_element_type=jnp.float32)

        def partial_l(c):
            load_x(c)
            return jnp.dot(x_vmem[...], w_vmem[:, h:n],
                           preferred_element_type=jnp.float32)

        send_r[...] = partial_r(lax.rem(r + N_DEV - 1, N_DEV)).astype(
            jnp.bfloat16)
        send_l[...] = partial_l(lax.rem(r + 1, N_DEV)).astype(jnp.bfloat16)

        def rs_step(s, carry):
            @pl.when(jnp.logical_or(j > 0, s > 0))
            def _():
                pl.semaphore_wait(credit_r, 1)
                pl.semaphore_wait(credit_l, 1)

            rdr = pltpu.make_async_remote_copy(
                src_ref=send_r, dst_ref=recv_r,
                send_sem=rsr_send.at[j, s], recv_sem=rsr_recv.at[j, s],
                device_id=(right,), device_id_type=pl.DeviceIdType.MESH)
            rdl = pltpu.make_async_remote_copy(
                src_ref=send_l, dst_ref=recv_l,
                send_sem=rsl_send.at[j, s], recv_sem=rsl_recv.at[j, s],
                device_id=(left,), device_id_type=pl.DeviceIdType.MESH)
            rdr.start()
            rdl.start()
            tmp_r[...] = partial_r(lax.rem(r - 2 - s + 2 * N_DEV, N_DEV))
            tmp_l[...] = partial_l(lax.rem(r + 2 + s, N_DEV))
            rdr.wait()
            rdl.wait()
            send_r[...] = (tmp_r[...] + recv_r[...].astype(
                jnp.float32)).astype(jnp.bfloat16)
            send_l[...] = (tmp_l[...] + recv_l[...].astype(
                jnp.float32)).astype(jnp.bfloat16)

            @pl.when(jnp.logical_not(
                jnp.logical_and(j == NB - 1, s == N_DEV - 2)))
            def _():
                pl.semaphore_signal(credit_r, inc=1, device_id=(left,),
                                    device_id_type=pl.DeviceIdType.MESH)
                pl.semaphore_signal(credit_l, inc=1, device_id=(right,),
                                    device_id_type=pl.DeviceIdType.MESH)
            return carry

        lax.fori_loop(0, N_DEV - 1, rs_step, 0)

        rows = pl.ds(r * chunk + j * band, band)
        tmp_r[...] = send_r[...].astype(jnp.float32)
        c1 = pltpu.make_async_copy(tmp_r, out_hbm.at[rows, 0:h], sem_a)
        c1.start()
        tmp_l[...] = send_l[...].astype(jnp.float32)
        c2 = pltpu.make_async_copy(tmp_l, out_hbm.at[rows, h:n], sem_b)
        c2.start()
        c1.wait()
        c2.wait()

        for t in range(N_DEV - 1):
            src_r = send_r if t == 0 else ag_r.at[t - 1]
            src_l = send_l if t == 0 else ag_l.at[t - 1]
            rdr = pltpu.make_async_remote_copy(
                src_ref=src_r, dst_ref=ag_r.at[t],
                send_sem=agr_send.at[j, t], recv_sem=agr_recv.at[j, t],
                device_id=(right,), device_id_type=pl.DeviceIdType.MESH)
            rdl = pltpu.make_async_remote_copy(
                src_ref=src_l, dst_ref=ag_l.at[t],
                send_sem=agl_send.at[j, t], recv_sem=agl_recv.at[j, t],
                device_id=(left,), device_id_type=pl.DeviceIdType.MESH)
            rdr.start()
            rdl.start()
            rdr.wait()
            rdl.wait()
            ar = lax.rem(r - 1 - t + N_DEV, N_DEV)
            rows_r = pl.ds(ar * chunk + j * band, band)
            tmp_r[...] = ag_r[t].astype(jnp.float32)
            s1 = pltpu.make_async_copy(tmp_r, out_hbm.at[rows_r, 0:h],
                                       sem_a)
            s1.start()
            al = lax.rem(r + 1 + t, N_DEV)
            rows_l = pl.ds(al * chunk + j * band, band)
            tmp_l[...] = ag_l[t].astype(jnp.float32)
            s2 = pltpu.make_async_copy(tmp_l, out_hbm.at[rows_l, h:n],
                                       sem_b)
            s2.start()
            s1.wait()
            s2.wait()

    nsteps = (NB, N_DEV - 1)
    return pl.pallas_call(
        body,
        grid=(NB,),
        out_shape=jax.ShapeDtypeStruct((m, n), jnp.float32),
        in_specs=[pl.BlockSpec(memory_space=pl.ANY),
                  pl.BlockSpec(memory_space=pl.ANY)],
        out_specs=pl.BlockSpec(memory_space=pl.ANY),
        scratch_shapes=[
            pltpu.VMEM((k_shard, n), jnp.bfloat16),
            pltpu.VMEM((band, k_shard), jnp.bfloat16),
            pltpu.VMEM((band, h), jnp.bfloat16),
            pltpu.VMEM((band, h), jnp.bfloat16),
            pltpu.VMEM((band, h), jnp.bfloat16),
            pltpu.VMEM((band, h), jnp.bfloat16),
            pltpu.VMEM((band, h), jnp.float32),
            pltpu.VMEM((band, h), jnp.float32),
            pltpu.VMEM((N_DEV - 1, band, h), jnp.bfloat16),
            pltpu.VMEM((N_DEV - 1, band, h), jnp.bfloat16),
            pltpu.SemaphoreType.DMA,
            pltpu.SemaphoreType.DMA,
            pltpu.SemaphoreType.DMA(nsteps),
            pltpu.SemaphoreType.DMA(nsteps),
            pltpu.SemaphoreType.DMA(nsteps),
            pltpu.SemaphoreType.DMA(nsteps),
            pltpu.SemaphoreType.DMA(nsteps),
            pltpu.SemaphoreType.DMA(nsteps),
            pltpu.SemaphoreType.DMA(nsteps),
            pltpu.SemaphoreType.DMA(nsteps),
            pltpu.SemaphoreType.REGULAR,
            pltpu.SemaphoreType.REGULAR,
        ],
        compiler_params=pltpu.CompilerParams(
            collective_id=0, vmem_limit_bytes=63 * 1024 * 1024,
            dimension_semantics=("arbitrary",)),
    )(x, w_mat)


# device time: 818837 ns/iter; 1.7663x vs baseline; 1.0333x over previous
import jax
import jax.numpy as jnp
from jax import lax
from jax.experimental import pallas as pl
from jax.experimental.pallas import tpu as pltpu

jax.config.update("jax_compilation_cache_dir", "/tmp/jaxcache")
jax.config.update("jax_persistent_cache_min_compile_time_secs", 1.0)

N_DEV = 4
NB = 4


def kernel(x, w_mat):
    x = x.astype(jnp.bfloat16)
    w_mat = w_mat.astype(jnp.bfloat16)
    m, k_shard = x.shape
    _, n = w_mat.shape
    chunk = m // N_DEV
    band = chunk // NB
    h = n // 2

    def body(x_hbm, w_hbm, out_hbm, w_vmem, x_vmem,
             send_r, recv_r, send_l, recv_l, tmp_r, tmp_l, ag_r, ag_l,
             sem_a, sem_b,
             rsr_send, rsr_recv, rsl_send, rsl_recv,
             agr_send, agr_recv, agl_send, agl_recv,
             credit_r, credit_l):
        j = pl.program_id(0)
        r = lax.axis_index("i")
        right = lax.rem(r + 1, N_DEV)
        left = lax.rem(r + N_DEV - 1, N_DEV)

        @pl.when(j == 0)
        def _():
            bsem = pltpu.get_barrier_semaphore()
            pl.semaphore_signal(bsem, inc=1, device_id=(left,),
                                device_id_type=pl.DeviceIdType.MESH)
            pl.semaphore_signal(bsem, inc=1, device_id=(right,),
                                device_id_type=pl.DeviceIdType.MESH)
            pl.semaphore_wait(bsem, 2)
            cw = pltpu.make_async_copy(w_hbm, w_vmem, sem_a)
            cw.start()
            cw.wait()

        def load_x(c):
            cp = pltpu.make_async_copy(
                x_hbm.at[pl.ds(c * chunk + j * band, band), :], x_vmem,
                sem_a)
            cp.start()
            cp.wait()

        def partial_r(c):
            load_x(c)
            return jnp.dot(x_vmem[...], w_vmem[:, 0:h],
                           preferred_element_type=jnp.float32)

        def partial_l(c):
            load_x(c)
            return jnp.dot(x_vmem[...], w_vmem[:, h:n],
                           preferred_element_type=jnp.float32)

        send_r[...] = partial_r(lax.rem(r + N_DEV - 1, N_DEV)).astype(
            jnp.bfloat16)
        send_l[...] = partial_l(lax.rem(r + 1, N_DEV)).astype(jnp.bfloat16)

        def rs_step(s, carry):
            @pl.when(jnp.logical_or(j > 0, s > 0))
            def _():
                pl.semaphore_wait(credit_r, 1)
                pl.semaphore_wait(credit_l, 1)

            rdr = pltpu.make_async_remote_copy(
                src_ref=send_r, dst_ref=recv_r,
                send_sem=rsr_send.at[j, s], recv_sem=rsr_recv.at[j, s],
                device_id=(right,), device_id_type=pl.DeviceIdType.MESH)
            rdl = pltpu.make_async_remote_copy(
                src_ref=send_l, dst_ref=recv_l,
                send_sem=rsl_send.at[j, s], recv_sem=rsl_recv.at[j, s],
                device_id=(left,), device_id_type=pl.DeviceIdType.MESH)
            rdr.start()
            rdl.start()
            tmp_r[...] = partial_r(lax.rem(r - 2 - s + 2 * N_DEV, N_DEV))
            tmp_l[...] = partial_l(lax.rem(r + 2 + s, N_DEV))
            rdr.wait()
            rdl.wait()
            send_r[...] = (tmp_r[...] + recv_r[...].astype(
                jnp.float32)).astype(jnp.bfloat16)
            send_l[...] = (tmp_l[...] + recv_l[...].astype(
                jnp.float32)).astype(jnp.bfloat16)

            @pl.when(jnp.logical_not(
                jnp.logical_and(j == NB - 1, s == N_DEV - 2)))
            def _():
                pl.semaphore_signal(credit_r, inc=1, device_id=(left,),
                                    device_id_type=pl.DeviceIdType.MESH)
                pl.semaphore_signal(credit_l, inc=1, device_id=(right,),
                                    device_id_type=pl.DeviceIdType.MESH)
            return carry

        lax.fori_loop(0, N_DEV - 1, rs_step, 0)

        rows = pl.ds(r * chunk + j * band, band)
        tmp_r[...] = send_r[...].astype(jnp.float32)
        c1 = pltpu.make_async_copy(tmp_r, out_hbm.at[rows, 0:h], sem_a)
        c1.start()
        tmp_l[...] = send_l[...].astype(jnp.float32)
        c2 = pltpu.make_async_copy(tmp_l, out_hbm.at[rows, h:n], sem_b)
        c2.start()
        c1.wait()
        c2.wait()

        def store_hop(t):
            ar = lax.rem(r - 1 - t + N_DEV, N_DEV)
            rows_r = pl.ds(ar * chunk + j * band, band)
            tmp_r[...] = ag_r[t].astype(jnp.float32)
            s1 = pltpu.make_async_copy(tmp_r, out_hbm.at[rows_r, 0:h],
                                       sem_a)
            s1.start()
            al = lax.rem(r + 1 + t, N_DEV)
            rows_l = pl.ds(al * chunk + j * band, band)
            tmp_l[...] = ag_l[t].astype(jnp.float32)
            s2 = pltpu.make_async_copy(tmp_l, out_hbm.at[rows_l, h:n],
                                       sem_b)
            s2.start()
            s1.wait()
            s2.wait()

        for t in range(N_DEV - 1):
            src_r = send_r if t == 0 else ag_r.at[t - 1]
            src_l = send_l if t == 0 else ag_l.at[t - 1]
            rdr = pltpu.make_async_remote_copy(
                src_ref=src_r, dst_ref=ag_r.at[t],
                send_sem=agr_send.at[j, t], recv_sem=agr_recv.at[j, t],
                device_id=(right,), device_id_type=pl.DeviceIdType.MESH)
            rdl = pltpu.make_async_remote_copy(
                src_ref=src_l, dst_ref=ag_l.at[t],
                send_sem=agl_send.at[j, t], recv_sem=agl_recv.at[j, t],
                device_id=(left,), device_id_type=pl.DeviceIdType.MESH)
            rdr.start()
            rdl.start()
            if t > 0:
                store_hop(t - 1)
            rdr.wait()
            rdl.wait()
        store_hop(N_DEV - 2)

    nsteps = (NB, N_DEV - 1)
    return pl.pallas_call(
        body,
        grid=(NB,),
        out_shape=jax.ShapeDtypeStruct((m, n), jnp.float32),
        in_specs=[pl.BlockSpec(memory_space=pl.ANY),
                  pl.BlockSpec(memory_space=pl.ANY)],
        out_specs=pl.BlockSpec(memory_space=pl.ANY),
        scratch_shapes=[
            pltpu.VMEM((k_shard, n), jnp.bfloat16),
            pltpu.VMEM((band, k_shard), jnp.bfloat16),
            pltpu.VMEM((band, h), jnp.bfloat16),
            pltpu.VMEM((band, h), jnp.bfloat16),
            pltpu.VMEM((band, h), jnp.bfloat16),
            pltpu.VMEM((band, h), jnp.bfloat16),
            pltpu.VMEM((band, h), jnp.float32),
            pltpu.VMEM((band, h), jnp.float32),
            pltpu.VMEM((N_DEV - 1, band, h), jnp.bfloat16),
            pltpu.VMEM((N_DEV - 1, band, h), jnp.bfloat16),
            pltpu.SemaphoreType.DMA,
            pltpu.SemaphoreType.DMA,
            pltpu.SemaphoreType.DMA(nsteps),
            pltpu.SemaphoreType.DMA(nsteps),
            pltpu.SemaphoreType.DMA(nsteps),
            pltpu.SemaphoreType.DMA(nsteps),
            pltpu.SemaphoreType.DMA(nsteps),
            pltpu.SemaphoreType.DMA(nsteps),
            pltpu.SemaphoreType.DMA(nsteps),
            pltpu.SemaphoreType.DMA(nsteps),
            pltpu.SemaphoreType.REGULAR,
            pltpu.SemaphoreType.REGULAR,
        ],
        compiler_params=pltpu.CompilerParams(
            collective_id=0, vmem_limit_bytes=63 * 1024 * 1024,
            dimension_semantics=("arbitrary",)),
    )(x, w_mat)
